# baseline (speedup 1.0000x reference)
import functools

import jax
import jax.numpy as jnp
from jax import lax
from jax.experimental import pallas as pl
from jax.experimental.pallas import tpu as pltpu
from jax.experimental.pallas import tpu_sc as plsc

_D = 64
_E = 8
_B = 1024
_S = 200

_NC = 2
_NS = 16
_NW = _NC * _NS
_BPW = _B // _NW
_BLK = 8
_NBLK = _BPW // _BLK
_NG = _S // 16
_TAIL = _S - 16


def _body(wid_hbm, en_hbm, tab2_hbm, out_hbm,
          idx_v, eni_v, idx2_v, rows_v, out_v, sem_r):
    w = lax.axis_index("s") * _NC + lax.axis_index("c")
    tile_base = w * _BPW
    lane = lax.iota(jnp.int32, 16)

    def block(bi, carry):
        b0 = pl.multiple_of(tile_base + bi * _BLK, 8)
        pltpu.sync_copy(wid_hbm.at[pl.ds(b0, _BLK)], idx_v)
        pltpu.sync_copy(en_hbm.at[pl.ds(b0, _BLK)], eni_v)

        def halve(rr, c2):
            for gi in range(_NG):
                off = gi * 16
                idx2_v[rr, pl.ds(off, 16)] = (
                    lax.shift_right_logical(idx_v[rr, pl.ds(off, 16)], 1))
            idx2_v[rr, pl.ds(_TAIL, 16)] = (
                lax.shift_right_logical(idx_v[rr, pl.ds(_TAIL, 16)], 1))
            return c2

        lax.fori_loop(0, _BLK, halve, 0)

        def row(rr, carry2):
            h1 = pltpu.async_copy(
                tab2_hbm.at[idx2_v.at[rr, pl.ds(0, 128)]],
                rows_v.at[pl.ds(0, 128)],
                sem_r)
            h2 = pltpu.async_copy(
                tab2_hbm.at[idx2_v.at[rr, pl.ds(128, _S - 128)]],
                rows_v.at[pl.ds(128, _S - 128)],
                sem_r)
            h1.wait()
            h2.wait()

            def assemble(off):
                vv = idx_v[rr, pl.ds(off, 16)]
                ev = eni_v[rr, pl.ds(off, 16)].astype(jnp.float32)
                for u in range(16):
                    r = off + u
                    half = (vv[u] & 1) * _D
                    for c in range(_D // 16):
                        out_v[r, pl.ds(c * 16, 16)] = (
                            rows_v[r, pl.ds(half + c * 16, 16)])
                    tail = out_v[r, pl.ds(_D - 8, 16)]
                    out_v[r, pl.ds(_D - 8, 16)] = jnp.where(
                        lane < 8, tail, ev[u])

            def assemble_g(gi, c3):
                assemble(gi * 16)
                return c3

            lax.fori_loop(0, _NG, assemble_g, 0)
            assemble(_TAIL)
            pltpu.sync_copy(out_v, out_hbm.at[b0 + rr])
            return carry2

        lax.fori_loop(0, _BLK, row, 0)
        return carry

    lax.fori_loop(0, _NBLK, block, 0)


@jax.jit
def _run(wid, en, table):
    tab2 = table.reshape(500000, 2 * _D)
    mesh = plsc.VectorSubcoreMesh(core_axis_name="c", subcore_axis_name="s")
    f = functools.partial(
        pl.kernel,
        mesh=mesh,
        out_type=jax.ShapeDtypeStruct((_B, _S, _D + _E), jnp.float32),
        scratch_types=[
            pltpu.VMEM((_BLK, _S), jnp.int32),
            pltpu.VMEM((_BLK, _S), jnp.int32),
            pltpu.VMEM((_BLK, _S), jnp.int32),
            pltpu.VMEM((_S, 2 * _D), jnp.float32),
            pltpu.VMEM((_S, _D + _E), jnp.float32),
            pltpu.SemaphoreType.DMA,
        ],
    )(_body)
    return f(wid, en, tab2)


def kernel(word_id, en_indicator, table):
    return _run(word_id, en_indicator, table)


# speedup vs baseline: 1.8007x; 1.8007x over previous
import functools

import jax
import jax.numpy as jnp
from jax import lax
from jax.experimental import pallas as pl
from jax.experimental.pallas import tpu as pltpu
from jax.experimental.pallas import tpu_sc as plsc

_D = 64
_E = 8
_B = 1024
_S = 200

_NC = 2
_NS = 16
_NW = _NC * _NS
_BPW = _B // _NW
_BLK = 8
_NBLK = _BPW // _BLK
_NG = _S // 16
_TAIL = _S - 16
_FIRED = _NG * 16 + 16


def _body(wid_hbm, en_hbm, table_hbm, out_hbm,
          idx_v, eni_v, out_a, out_b, drain_v,
          sem_ra, sem_rb, sem_wa, sem_wb):
    w = lax.axis_index("s") * _NC + lax.axis_index("c")
    tile_base = w * _BPW
    lane = lax.iota(jnp.int32, 16)
    obufs = [out_a, out_b]
    rsems = [sem_ra, sem_rb]
    wsems = [sem_wa, sem_wb]

    def fire(rr, p):
        out_v = obufs[p]

        def fire_off(off):
            ivec = idx_v[rr, pl.ds(off, 16)]
            for u in range(16):
                pltpu.async_copy(
                    table_hbm.at[ivec[u]],
                    out_v.at[off + u, pl.ds(0, _D)],
                    rsems[p])

        def fire_g(gi, c3):
            fire_off(gi * 16)
            return c3

        lax.fori_loop(0, _NG, fire_g, 0)
        fire_off(_TAIL)

    def drain(p):
        pltpu.make_async_copy(
            wid_hbm.at[pl.ds(0, _FIRED * _D // 128), pl.ds(0, 128)],
            drain_v,
            rsems[p]).wait()

    def blend_row(rr, p):
        out_v = obufs[p]

        def blend(off):
            ev = eni_v[rr, pl.ds(off, 16)].astype(jnp.float32)
            for u in range(16):
                r = off + u
                tail = out_v[r, pl.ds(_D - 8, 16)]
                out_v[r, pl.ds(_D - 8, 16)] = jnp.where(lane < 8, tail, ev[u])

        def blend_g(gi, c3):
            blend(gi * 16)
            return c3

        lax.fori_loop(0, _NG, blend_g, 0)
        blend(_TAIL)

    def block(bi, carry):
        b0 = pl.multiple_of(tile_base + bi * _BLK, 8)
        pltpu.sync_copy(wid_hbm.at[pl.ds(b0, _BLK)], idx_v)
        pltpu.sync_copy(en_hbm.at[pl.ds(b0, _BLK)], eni_v)

        fire(0, 0)
        whs = [None, None]
        for rr in range(_BLK):
            p = rr % 2
            if rr + 1 < _BLK:
                if whs[1 - p] is not None:
                    whs[1 - p].wait()
                fire(rr + 1, 1 - p)
            drain(p)
            blend_row(rr, p)
            whs[p] = pltpu.async_copy(obufs[p], out_hbm.at[b0 + rr], wsems[p])
        whs[0].wait()
        whs[1].wait()
        return carry

    lax.fori_loop(0, _NBLK, block, 0)


@jax.jit
def _run(wid, en, table):
    mesh = plsc.VectorSubcoreMesh(core_axis_name="c", subcore_axis_name="s")
    f = functools.partial(
        pl.kernel,
        mesh=mesh,
        out_type=jax.ShapeDtypeStruct((_B, _S, _D + _E), jnp.float32),
        scratch_types=[
            pltpu.VMEM((_BLK, _S), jnp.int32),
            pltpu.VMEM((_BLK, _S), jnp.int32),
            pltpu.VMEM((_S, _D + _E), jnp.float32),
            pltpu.VMEM((_S, _D + _E), jnp.float32),
            pltpu.VMEM((_FIRED * _D // 128, 128), jnp.int32),
            pltpu.SemaphoreType.DMA,
            pltpu.SemaphoreType.DMA,
            pltpu.SemaphoreType.DMA,
            pltpu.SemaphoreType.DMA,
        ],
    )(_body)
    return f(wid, en, table)


def kernel(word_id, en_indicator, table):
    return _run(word_id, en_indicator, table)
